# Initial kernel scaffold; baseline (speedup 1.0000x reference)
import jax
import jax.numpy as jnp
from jax.experimental import pallas as pl


def kernel(x, edge_index, W1, b1, W_mu, b_mu, W_ls, b_ls):
    raise NotImplementedError("write your pallas kernel here")


# speedup vs baseline: 2.3325x; 2.3325x over previous
import functools
import jax
import jax.numpy as jnp
from jax import lax
from jax.experimental import pallas as pl
from jax.experimental.pallas import tpu as pltpu

N = 10000
D_IN = 256
D_HID = 512
D_OUT = 256
MB = 1000
GRID = N // MB
NCHUNK = 4
CW = 128


def _dot(a, b):
    return lax.dot_general(a, b, (((1,), (0,)), ((), ())),
                           preferred_element_type=jnp.float32,
                           precision=lax.Precision.HIGHEST)


def _stage_a_body(x_ref, w1_ref, degp_ref, gs_ref, r_ref):
    deg = jnp.sum(degp_ref[0], axis=0) + 1.0
    r = lax.rsqrt(deg)
    g = _dot(x_ref[...], w1_ref[...])
    gs = g * r[:, None]
    gs_ref[...] = gs.reshape(MB, NCHUNK, CW).transpose(1, 0, 2)
    r_ref[...] = r[None, None, :]


def _stage_a(x, w1, degp):
    p = degp.shape[1]
    return pl.pallas_call(
        _stage_a_body,
        grid=(GRID,),
        in_specs=[
            pl.BlockSpec((MB, D_IN), lambda i: (i, 0)),
            pl.BlockSpec((D_IN, D_HID), lambda i: (0, 0)),
            pl.BlockSpec((1, p, MB), lambda i: (i, 0, 0)),
        ],
        out_specs=[
            pl.BlockSpec((NCHUNK, MB, CW), lambda i: (0, i, 0)),
            pl.BlockSpec((1, 1, MB), lambda i: (i, 0, 0)),
        ],
        out_shape=[
            jax.ShapeDtypeStruct((NCHUNK, N, CW), jnp.float32),
            jax.ShapeDtypeStruct((GRID, 1, MB), jnp.float32),
        ],
    )(x, w1, degp)


def _stage_b_body(t1_ref, r_ref, b1_ref, wmu_ref, wls_ref, ps_ref):
    t = t1_ref[...].transpose(1, 0, 2).reshape(MB, D_HID)
    r = r_ref[0, 0, :]
    h = jnp.maximum(t * r[:, None] + b1_ref[0, :][None, :], 0.0)
    p1 = _dot(h, wmu_ref[...]) * r[:, None]
    p2 = _dot(h, wls_ref[...]) * r[:, None]
    ps = jnp.concatenate([p1, p2], axis=1)
    ps_ref[...] = ps.reshape(MB, NCHUNK, CW).transpose(1, 0, 2)


def _stage_b(t1, r, b1, wmu, wls):
    return pl.pallas_call(
        _stage_b_body,
        grid=(GRID,),
        in_specs=[
            pl.BlockSpec((NCHUNK, MB, CW), lambda i: (0, i, 0)),
            pl.BlockSpec((1, 1, MB), lambda i: (i, 0, 0)),
            pl.BlockSpec((1, D_HID), lambda i: (0, 0)),
            pl.BlockSpec((D_HID, D_OUT), lambda i: (0, 0)),
            pl.BlockSpec((D_HID, D_OUT), lambda i: (0, 0)),
        ],
        out_specs=pl.BlockSpec((NCHUNK, MB, CW), lambda i: (0, i, 0)),
        out_shape=jax.ShapeDtypeStruct((NCHUNK, N, CW), jnp.float32),
    )(t1, r, b1, wmu, wls)


def _stage_c_body(t2_ref, r_ref, bmu_ref, bls_ref, mu_ref, ls_ref):
    t = t2_ref[...].transpose(1, 0, 2).reshape(MB, 2 * D_OUT)
    r = r_ref[0, 0, :]
    mu_ref[...] = t[:, :D_OUT] * r[:, None] + bmu_ref[0, :][None, :]
    ls_ref[...] = t[:, D_OUT:] * r[:, None] + bls_ref[0, :][None, :]


def _stage_c(t2, r, bmu, bls):
    return pl.pallas_call(
        _stage_c_body,
        grid=(GRID,),
        in_specs=[
            pl.BlockSpec((NCHUNK, MB, CW), lambda i: (0, i, 0)),
            pl.BlockSpec((1, 1, MB), lambda i: (i, 0, 0)),
            pl.BlockSpec((1, D_OUT), lambda i: (0, 0)),
            pl.BlockSpec((1, D_OUT), lambda i: (0, 0)),
        ],
        out_specs=[
            pl.BlockSpec((MB, D_OUT), lambda i: (i, 0)),
            pl.BlockSpec((MB, D_OUT), lambda i: (i, 0)),
        ],
        out_shape=[
            jax.ShapeDtypeStruct((N, D_OUT), jnp.float32),
            jax.ShapeDtypeStruct((N, D_OUT), jnp.float32),
        ],
    )(t2, r, bmu, bls)


def _spmm_glue(gs, src, dst):
    nm = gs.transpose(1, 0, 2).reshape(N, D_HID)
    t = nm + jnp.zeros_like(nm).at[dst].add(nm[src])
    return t.reshape(N, NCHUNK, CW).transpose(1, 0, 2)


def kernel(x, edge_index, W1, b1, W_mu, b_mu, W_ls, b_ls):
    src = edge_index[0].astype(jnp.int32)
    dst = edge_index[1].astype(jnp.int32)

    deg = jnp.zeros((N,), jnp.float32).at[dst].add(1.0)
    degp = deg.reshape(GRID, 1, MB)

    gs, r = _stage_a(x, W1, degp)
    t1 = _spmm_glue(gs, src, dst)
    ps = _stage_b(t1, r, b1.reshape(1, -1), W_mu, W_ls)
    t2 = _spmm_glue(ps, src, dst)
    mu, ls = _stage_c(t2, r, b_mu.reshape(1, -1), b_ls.reshape(1, -1))
    return (mu, ls)


# speedup vs baseline: 5.7423x; 2.4618x over previous
import functools
import jax
import jax.numpy as jnp
from jax import lax
from jax.experimental import pallas as pl
from jax.experimental.pallas import tpu as pltpu
from jax.experimental.pallas import tpu_sc as plsc

N = 10000
D_IN = 256
D_HID = 512
D_OUT = 256
MB = 1000
GRID = N // MB
NCHUNK = 4
CW = 128

E = 160000
EPAD = 163840
EPT = EPAD // 16
NB = EPT // 128
ACC_ROWS = 10008
ST_A = 632
ST_LAST = N - 15 * ST_A


def _striped_copy(s, mk_src, mk_dst, total=N):
    last = total - 15 * ST_A

    @pl.when(s < 15)
    def _():
        off = pl.multiple_of(s * ST_A, 8)
        pltpu.sync_copy(mk_src(off, ST_A), mk_dst(off, ST_A))

    @pl.when(s == 15)
    def _():
        pltpu.sync_copy(mk_src(15 * ST_A, last), mk_dst(15 * ST_A, last))

_SC_MESH = dict(core_axis_name="c", subcore_axis_name="s",
                num_cores=2, num_subcores=16)


def _dot(a, b):
    return lax.dot_general(a, b, (((1,), (0,)), ((), ())),
                           preferred_element_type=jnp.float32,
                           precision=lax.Precision.HIGHEST)


def _stage_a_body(x_ref, w1_ref, degp_ref, gs_ref, r_ref):
    deg = degp_ref[0, :, 0] + degp_ref[1, :, 0] + 1.0
    r = lax.rsqrt(deg)
    g = _dot(x_ref[...], w1_ref[...])
    gs = g * r[:, None]
    gs_ref[...] = gs.reshape(MB, NCHUNK, CW).transpose(1, 0, 2)
    r_ref[...] = r[None, None, :]


def _stage_a(x, w1, degp):
    return pl.pallas_call(
        _stage_a_body,
        grid=(GRID,),
        in_specs=[
            pl.BlockSpec((MB, D_IN), lambda i: (i, 0)),
            pl.BlockSpec((D_IN, D_HID), lambda i: (0, 0)),
            pl.BlockSpec((2, MB, 128), lambda i: (0, i, 0)),
        ],
        out_specs=[
            pl.BlockSpec((NCHUNK, MB, CW), lambda i: (0, i, 0)),
            pl.BlockSpec((1, 1, MB), lambda i: (i, 0, 0)),
        ],
        out_shape=[
            jax.ShapeDtypeStruct((NCHUNK, N, CW), jnp.float32),
            jax.ShapeDtypeStruct((GRID, 1, MB), jnp.float32),
        ],
    )(x, w1, degp)


def _stage_b_body(t1_ref, r_ref, b1_ref, wmu_ref, wls_ref, ps_ref):
    t = t1_ref[...].transpose(1, 0, 2).reshape(MB, D_HID)
    r = r_ref[0, 0, :]
    h = jnp.maximum(t * r[:, None] + b1_ref[0, :][None, :], 0.0)
    p1 = _dot(h, wmu_ref[...]) * r[:, None]
    p2 = _dot(h, wls_ref[...]) * r[:, None]
    ps = jnp.concatenate([p1, p2], axis=1)
    ps_ref[...] = ps.reshape(MB, NCHUNK, CW).transpose(1, 0, 2)


def _stage_b(t1, r, b1, wmu, wls):
    return pl.pallas_call(
        _stage_b_body,
        grid=(GRID,),
        in_specs=[
            pl.BlockSpec((NCHUNK, MB, CW), lambda i: (0, i, 0)),
            pl.BlockSpec((1, 1, MB), lambda i: (i, 0, 0)),
            pl.BlockSpec((1, D_HID), lambda i: (0, 0)),
            pl.BlockSpec((D_HID, D_OUT), lambda i: (0, 0)),
            pl.BlockSpec((D_HID, D_OUT), lambda i: (0, 0)),
        ],
        out_specs=pl.BlockSpec((NCHUNK, MB, CW), lambda i: (0, i, 0)),
        out_shape=jax.ShapeDtypeStruct((NCHUNK, N, CW), jnp.float32),
    )(t1, r, b1, wmu, wls)


def _stage_c_body(t2_ref, r_ref, bmu_ref, bls_ref, mu_ref, ls_ref):
    t = t2_ref[...].transpose(1, 0, 2).reshape(MB, 2 * D_OUT)
    r = r_ref[0, 0, :]
    mu_ref[...] = t[:, :D_OUT] * r[:, None] + bmu_ref[0, :][None, :]
    ls_ref[...] = t[:, D_OUT:] * r[:, None] + bls_ref[0, :][None, :]


def _stage_c(t2, r, bmu, bls):
    return pl.pallas_call(
        _stage_c_body,
        grid=(GRID,),
        in_specs=[
            pl.BlockSpec((NCHUNK, MB, CW), lambda i: (0, i, 0)),
            pl.BlockSpec((1, 1, MB), lambda i: (i, 0, 0)),
            pl.BlockSpec((1, D_OUT), lambda i: (0, 0)),
            pl.BlockSpec((1, D_OUT), lambda i: (0, 0)),
        ],
        out_specs=[
            pl.BlockSpec((MB, D_OUT), lambda i: (i, 0)),
            pl.BlockSpec((MB, D_OUT), lambda i: (i, 0)),
        ],
        out_shape=[
            jax.ShapeDtypeStruct((N, D_OUT), jnp.float32),
            jax.ShapeDtypeStruct((N, D_OUT), jnp.float32),
        ],
    )(t2, r, bmu, bls)


def _deg_sc(dst32, ones, zeros):
    @functools.partial(
        pl.kernel,
        out_type=jax.ShapeDtypeStruct((2, N, 128), jnp.float32),
        mesh=plsc.VectorSubcoreMesh(**_SC_MESH),
        scratch_types=[
            pltpu.VMEM((EPAD // 32 // 128, 128), jnp.int32),
            pltpu.VMEM((128, 128), jnp.float32),
            pltpu.VMEM_SHARED((10008, 128), jnp.float32),
            pltpu.SemaphoreType.DMA,
        ],
    )
    def run(dst_hbm, ones_hbm, zeros_hbm, degp_hbm, dst_v, ones_v, acc, sem):
        c = lax.axis_index("c")
        s = lax.axis_index("s")
        w = c * 16 + s
        pltpu.sync_copy(dst_hbm.at[w], dst_v)
        pltpu.sync_copy(ones_hbm, ones_v)
        _striped_copy(s, lambda o, l: zeros_hbm.at[pl.ds(o, l)],
                      lambda o, l: acc.at[pl.ds(o, l)], total=10008)
        plsc.subcore_barrier()

        def body(b, carry):
            pltpu.async_copy(ones_v, acc.at[dst_v.at[b]], sem, add=True).wait()
            return carry
        lax.fori_loop(0, EPAD // 32 // 128, body, 0)

        plsc.subcore_barrier()
        _striped_copy(s, lambda o, l: acc.at[pl.ds(o, l)],
                      lambda o, l: degp_hbm.at[c, pl.ds(o, l)])

    return run(dst32, ones, zeros)


def _spmm_sc(table2d, src_flat, dst16):
    @functools.partial(
        pl.kernel,
        out_type=jax.ShapeDtypeStruct((NCHUNK * N, CW), jnp.float32),
        mesh=plsc.VectorSubcoreMesh(**_SC_MESH),
        scratch_types=[
            pltpu.VMEM((EPT,), jnp.int32),
            pltpu.VMEM((NB, 128), jnp.int32),
            pltpu.VMEM((128, CW), jnp.float32),
            pltpu.VMEM_SHARED((ACC_ROWS, CW), jnp.float32),
            pltpu.SemaphoreType.DMA,
            pltpu.SemaphoreType.DMA,
        ],
    )
    def run(tab_hbm, src_hbm, dst_hbm, out_hbm, src_v, dst_v, rows, acc,
            gsem, ssem):
        c = lax.axis_index("c")
        s = lax.axis_index("s")
        pltpu.sync_copy(src_hbm.at[pl.ds(s * EPT, EPT)], src_v)
        pltpu.sync_copy(dst_hbm.at[s], dst_v)

        for p in range(2):
            delta = c * (2 * N) if p == 0 else N

            def adj(i, carry):
                ib = pl.multiple_of(i * 16, 16)
                src_v[pl.ds(ib, 16)] = src_v[pl.ds(ib, 16)] + delta
                return carry
            lax.fori_loop(0, EPT // 16, adj, 0)

            chunk = c * 2 + p
            base = chunk * N
            _striped_copy(
                s,
                lambda o, l: tab_hbm.at[pl.ds(pl.multiple_of(base + o, 8), l)],
                lambda o, l: acc.at[pl.ds(o, l)])
            plsc.subcore_barrier()

            def bat(b, carry):
                bb = pl.multiple_of(b * 128, 128)
                pltpu.async_copy(tab_hbm.at[src_v.at[pl.ds(bb, 128)]],
                                 rows, gsem).wait()
                pltpu.async_copy(rows, acc.at[dst_v.at[b]], ssem,
                                 add=True).wait()
                return carry
            lax.fori_loop(0, NB, bat, 0)

            plsc.subcore_barrier()
            _striped_copy(
                s,
                lambda o, l: acc.at[pl.ds(o, l)],
                lambda o, l: out_hbm.at[pl.ds(pl.multiple_of(base + o, 8), l)])

    return run(table2d, src_flat, dst16)


def kernel(x, edge_index, W1, b1, W_mu, b_mu, W_ls, b_ls):
    src = edge_index[0].astype(jnp.int32)
    dst = edge_index[1].astype(jnp.int32)
    src_pad = jnp.concatenate([src, jnp.zeros((EPAD - E,), jnp.int32)])
    dst_pad = jnp.concatenate([dst, jnp.full((EPAD - E,), N, jnp.int32)])
    dst32 = dst_pad.reshape(32, EPAD // 32 // 128, 128)
    dst16 = dst_pad.reshape(16, NB, 128)
    ones = jnp.ones((128, 128), jnp.float32)
    zeros = jnp.zeros((10008, 128), jnp.float32)

    degp = _deg_sc(dst32, ones, zeros)
    gs, r = _stage_a(x, W1, degp)
    t1 = _spmm_sc(gs.reshape(NCHUNK * N, CW), src_pad, dst16)
    ps = _stage_b(t1.reshape(NCHUNK, N, CW), r, b1.reshape(1, -1),
                  W_mu, W_ls)
    t2 = _spmm_sc(ps.reshape(NCHUNK * N, CW), src_pad, dst16)
    mu, ls = _stage_c(t2.reshape(NCHUNK, N, CW), r, b_mu.reshape(1, -1),
                      b_ls.reshape(1, -1))
    return (mu, ls)


# speedup vs baseline: 6.9402x; 1.2086x over previous
import functools
import jax
import jax.numpy as jnp
from jax import lax
from jax.experimental import pallas as pl
from jax.experimental.pallas import tpu as pltpu
from jax.experimental.pallas import tpu_sc as plsc

N = 10000
D_IN = 256
D_HID = 512
D_OUT = 256
MB = 1000
GRID = N // MB
NCHUNK = 4
CW = 128

E = 160000
EPAD = 163840
EPT = EPAD // 16
NB = EPT // 128
SEG = 16
ACC_ROWS = 10008
ST_A = 632
ST_LAST = N - 15 * ST_A


def _striped_copy(s, mk_src, mk_dst, total=N):
    last = total - 15 * ST_A

    @pl.when(s < 15)
    def _():
        off = pl.multiple_of(s * ST_A, 8)
        pltpu.sync_copy(mk_src(off, ST_A), mk_dst(off, ST_A))

    @pl.when(s == 15)
    def _():
        pltpu.sync_copy(mk_src(15 * ST_A, last), mk_dst(15 * ST_A, last))

_SC_MESH = dict(core_axis_name="c", subcore_axis_name="s",
                num_cores=2, num_subcores=16)


def _dot(a, b):
    return lax.dot_general(a, b, (((1,), (0,)), ((), ())),
                           preferred_element_type=jnp.float32,
                           precision=lax.Precision.HIGHEST)


def _stage_a_body(x_ref, w1_ref, degp_ref, gs_ref, r_ref):
    deg = degp_ref[0, :, 0] + degp_ref[1, :, 0] + 1.0
    r = lax.rsqrt(deg)
    g = _dot(x_ref[...], w1_ref[...])
    gs = g * r[:, None]
    gs_ref[...] = gs.reshape(MB, NCHUNK, CW).transpose(1, 0, 2)
    r_ref[...] = r[None, None, :]


def _stage_a(x, w1, degp):
    return pl.pallas_call(
        _stage_a_body,
        grid=(GRID,),
        in_specs=[
            pl.BlockSpec((MB, D_IN), lambda i: (i, 0)),
            pl.BlockSpec((D_IN, D_HID), lambda i: (0, 0)),
            pl.BlockSpec((2, MB, 128), lambda i: (0, i, 0)),
        ],
        out_specs=[
            pl.BlockSpec((NCHUNK, MB, CW), lambda i: (0, i, 0)),
            pl.BlockSpec((1, 1, MB), lambda i: (i, 0, 0)),
        ],
        out_shape=[
            jax.ShapeDtypeStruct((NCHUNK, N, CW), jnp.float32),
            jax.ShapeDtypeStruct((GRID, 1, MB), jnp.float32),
        ],
    )(x, w1, degp)


def _stage_b_body(t1_ref, r_ref, b1_ref, wmu_ref, wls_ref, ps_ref):
    t = t1_ref[...].transpose(1, 0, 2).reshape(MB, D_HID)
    r = r_ref[0, 0, :]
    h = jnp.maximum(t * r[:, None] + b1_ref[0, :][None, :], 0.0)
    p1 = _dot(h, wmu_ref[...]) * r[:, None]
    p2 = _dot(h, wls_ref[...]) * r[:, None]
    ps = jnp.concatenate([p1, p2], axis=1)
    ps_ref[...] = ps.reshape(MB, NCHUNK, CW).transpose(1, 0, 2)


def _stage_b(t1, r, b1, wmu, wls):
    return pl.pallas_call(
        _stage_b_body,
        grid=(GRID,),
        in_specs=[
            pl.BlockSpec((NCHUNK, MB, CW), lambda i: (0, i, 0)),
            pl.BlockSpec((1, 1, MB), lambda i: (i, 0, 0)),
            pl.BlockSpec((1, D_HID), lambda i: (0, 0)),
            pl.BlockSpec((D_HID, D_OUT), lambda i: (0, 0)),
            pl.BlockSpec((D_HID, D_OUT), lambda i: (0, 0)),
        ],
        out_specs=pl.BlockSpec((NCHUNK, MB, CW), lambda i: (0, i, 0)),
        out_shape=jax.ShapeDtypeStruct((NCHUNK, N, CW), jnp.float32),
    )(t1, r, b1, wmu, wls)


def _stage_c_body(t2_ref, r_ref, bmu_ref, bls_ref, mu_ref, ls_ref):
    t = t2_ref[...].transpose(1, 0, 2).reshape(MB, 2 * D_OUT)
    r = r_ref[0, 0, :]
    mu_ref[...] = t[:, :D_OUT] * r[:, None] + bmu_ref[0, :][None, :]
    ls_ref[...] = t[:, D_OUT:] * r[:, None] + bls_ref[0, :][None, :]


def _stage_c(t2, r, bmu, bls):
    return pl.pallas_call(
        _stage_c_body,
        grid=(GRID,),
        in_specs=[
            pl.BlockSpec((NCHUNK, MB, CW), lambda i: (0, i, 0)),
            pl.BlockSpec((1, 1, MB), lambda i: (i, 0, 0)),
            pl.BlockSpec((1, D_OUT), lambda i: (0, 0)),
            pl.BlockSpec((1, D_OUT), lambda i: (0, 0)),
        ],
        out_specs=[
            pl.BlockSpec((MB, D_OUT), lambda i: (i, 0)),
            pl.BlockSpec((MB, D_OUT), lambda i: (i, 0)),
        ],
        out_shape=[
            jax.ShapeDtypeStruct((N, D_OUT), jnp.float32),
            jax.ShapeDtypeStruct((N, D_OUT), jnp.float32),
        ],
    )(t2, r, bmu, bls)


def _deg_sc(dst32, ones, zeros):
    @functools.partial(
        pl.kernel,
        out_type=jax.ShapeDtypeStruct((2, N, 128), jnp.float32),
        mesh=plsc.VectorSubcoreMesh(**_SC_MESH),
        scratch_types=[
            pltpu.VMEM((EPAD // 32 // 128, 128), jnp.int32),
            pltpu.VMEM((128, 128), jnp.float32),
            pltpu.VMEM_SHARED((10008, 128), jnp.float32),
            pltpu.SemaphoreType.DMA,
        ],
    )
    def run(dst_hbm, ones_hbm, zeros_hbm, degp_hbm, dst_v, ones_v, acc, sem):
        c = lax.axis_index("c")
        s = lax.axis_index("s")
        w = c * 16 + s
        pltpu.sync_copy(dst_hbm.at[w], dst_v)
        pltpu.sync_copy(ones_hbm, ones_v)
        _striped_copy(s, lambda o, l: zeros_hbm.at[pl.ds(o, l)],
                      lambda o, l: acc.at[pl.ds(o, l)], total=10008)
        plsc.subcore_barrier()

        def body(b, carry):
            pltpu.async_copy(ones_v, acc.at[dst_v.at[b]], sem, add=True).wait()
            return carry
        lax.fori_loop(0, EPAD // 32 // 128, body, 0)

        plsc.subcore_barrier()
        _striped_copy(s, lambda o, l: acc.at[pl.ds(o, l)],
                      lambda o, l: degp_hbm.at[c, pl.ds(o, l)])

    return run(dst32, ones, zeros)


def _spmm_sc(table2d, src_flat, dst16):
    @functools.partial(
        pl.kernel,
        out_type=jax.ShapeDtypeStruct((NCHUNK * N, CW), jnp.float32),
        mesh=plsc.VectorSubcoreMesh(**_SC_MESH),
        scratch_types=[
            pltpu.VMEM((EPT,), jnp.int32),
            pltpu.VMEM((2, SEG, 128), jnp.int32),
            pltpu.VMEM((2, 128, CW), jnp.float32),
            pltpu.VMEM_SHARED((ACC_ROWS, CW), jnp.float32),
            pltpu.SemaphoreType.DMA((2,)),
            pltpu.SemaphoreType.DMA((2,)),
            pltpu.SemaphoreType.DMA((2,)),
        ],
    )
    def run(tab_hbm, src_hbm, dst_hbm, out_hbm, src_v, dst_v, rows, acc,
            gsem, ssem, dsem):
        c = lax.axis_index("c")
        s = lax.axis_index("s")
        pltpu.sync_copy(src_hbm.at[pl.ds(s * EPT, EPT)], src_v)

        def start_dseg(g):
            pltpu.async_copy(dst_hbm.at[s, pl.ds(g * SEG, SEG)],
                             dst_v.at[g % 2], dsem.at[g % 2])

        for p in range(2):
            delta = c * (2 * N) if p == 0 else N

            def adj(i, carry):
                ib = pl.multiple_of(i * 16, 16)
                src_v[pl.ds(ib, 16)] = src_v[pl.ds(ib, 16)] + delta
                return carry
            lax.fori_loop(0, EPT // 16, adj, 0)

            chunk = c * 2 + p
            base = chunk * N
            _striped_copy(
                s,
                lambda o, l: tab_hbm.at[pl.ds(pl.multiple_of(base + o, 8), l)],
                lambda o, l: acc.at[pl.ds(o, l)])
            plsc.subcore_barrier()

            def start_gather(b, r):
                bb = pl.multiple_of(b * 128, 128)
                pltpu.async_copy(tab_hbm.at[src_v.at[pl.ds(bb, 128)]],
                                 rows.at[r], gsem.at[r])

            start_dseg(0)
            start_dseg(1)
            start_gather(0, 0)
            start_gather(1, 1)
            for g in range(NB // SEG):
                q = g % 2
                pltpu.make_async_copy(dst_hbm.at[s, pl.ds(0, SEG)],
                                      dst_v.at[q], dsem.at[q]).wait()

                def inner(j2, carry):
                    for r in range(2):
                        jloc = j2 * 2 + r
                        b = g * SEG + jloc
                        pltpu.make_async_copy(
                            tab_hbm.at[pl.ds(0, 128)], rows.at[r],
                            gsem.at[r]).wait()
                        pltpu.async_copy(rows.at[r],
                                         acc.at[dst_v.at[q, jloc]],
                                         ssem.at[r], add=True).wait()

                        @pl.when(b + 2 < NB)
                        def _():
                            start_gather(b + 2, r)
                    return carry
                lax.fori_loop(0, SEG // 2, inner, 0)
                if g + 2 < NB // SEG:
                    start_dseg(g + 2)

            plsc.subcore_barrier()
            _striped_copy(
                s,
                lambda o, l: acc.at[pl.ds(o, l)],
                lambda o, l: out_hbm.at[pl.ds(pl.multiple_of(base + o, 8), l)])

    return run(table2d, src_flat, dst16)


def kernel(x, edge_index, W1, b1, W_mu, b_mu, W_ls, b_ls):
    src = edge_index[0].astype(jnp.int32)
    dst = edge_index[1].astype(jnp.int32)
    src_pad = jnp.concatenate([src, jnp.zeros((EPAD - E,), jnp.int32)])
    dst_pad = jnp.concatenate([dst, jnp.full((EPAD - E,), N, jnp.int32)])
    dst32 = dst_pad.reshape(32, EPAD // 32 // 128, 128)
    dst16 = dst_pad.reshape(16, NB, 128)
    ones = jnp.ones((128, 128), jnp.float32)
    zeros = jnp.zeros((10008, 128), jnp.float32)

    degp = _deg_sc(dst32, ones, zeros)
    gs, r = _stage_a(x, W1, degp)
    t1 = _spmm_sc(gs.reshape(NCHUNK * N, CW), src_pad, dst16)
    ps = _stage_b(t1.reshape(NCHUNK, N, CW), r, b1.reshape(1, -1),
                  W_mu, W_ls)
    t2 = _spmm_sc(ps.reshape(NCHUNK * N, CW), src_pad, dst16)
    mu, ls = _stage_c(t2.reshape(NCHUNK, N, CW), r, b_mu.reshape(1, -1),
                      b_ls.reshape(1, -1))
    return (mu, ls)


# speedup vs baseline: 7.2544x; 1.0453x over previous
import functools
import jax
import jax.numpy as jnp
from jax import lax
from jax.experimental import pallas as pl
from jax.experimental.pallas import tpu as pltpu
from jax.experimental.pallas import tpu_sc as plsc

N = 10000
D_IN = 256
D_HID = 512
D_OUT = 256
MB = 1000
GRID = N // MB
NCHUNK = 4
CW = 128

E = 160000
EPAD = 163840
EPT = EPAD // 16
BT = 64
NB = EPT // BT
SEG = 16
ACC_ROWS = 10008
ST_A = 632
ST_LAST = N - 15 * ST_A


def _striped_copy(s, mk_src, mk_dst, total=N):
    last = total - 15 * ST_A

    @pl.when(s < 15)
    def _():
        off = pl.multiple_of(s * ST_A, 8)
        pltpu.sync_copy(mk_src(off, ST_A), mk_dst(off, ST_A))

    @pl.when(s == 15)
    def _():
        pltpu.sync_copy(mk_src(15 * ST_A, last), mk_dst(15 * ST_A, last))

_SC_MESH = dict(core_axis_name="c", subcore_axis_name="s",
                num_cores=2, num_subcores=16)


def _dot(a, b):
    return lax.dot_general(a, b, (((1,), (0,)), ((), ())),
                           preferred_element_type=jnp.float32,
                           precision=lax.Precision.HIGHEST)


def _stage_a_body(x_ref, w1_ref, degp_ref, gs_ref, r_ref):
    deg = degp_ref[0, :, 0] + degp_ref[1, :, 0] + 1.0
    r = lax.rsqrt(deg)
    g = _dot(x_ref[...], w1_ref[...])
    gs = g * r[:, None]
    gs_ref[...] = gs.reshape(MB, NCHUNK, CW).transpose(1, 0, 2)
    r_ref[...] = r[None, None, :]


def _stage_a(x, w1, degp):
    return pl.pallas_call(
        _stage_a_body,
        grid=(GRID,),
        in_specs=[
            pl.BlockSpec((MB, D_IN), lambda i: (i, 0)),
            pl.BlockSpec((D_IN, D_HID), lambda i: (0, 0)),
            pl.BlockSpec((2, MB, 128), lambda i: (0, i, 0)),
        ],
        out_specs=[
            pl.BlockSpec((NCHUNK, MB, CW), lambda i: (0, i, 0)),
            pl.BlockSpec((1, 1, MB), lambda i: (i, 0, 0)),
        ],
        out_shape=[
            jax.ShapeDtypeStruct((NCHUNK, N, CW), jnp.float32),
            jax.ShapeDtypeStruct((GRID, 1, MB), jnp.float32),
        ],
    )(x, w1, degp)


def _stage_b_body(t1_ref, r_ref, b1_ref, wmu_ref, wls_ref, ps_ref):
    t = t1_ref[...].transpose(1, 0, 2).reshape(MB, D_HID)
    r = r_ref[0, 0, :]
    h = jnp.maximum(t * r[:, None] + b1_ref[0, :][None, :], 0.0)
    p1 = _dot(h, wmu_ref[...]) * r[:, None]
    p2 = _dot(h, wls_ref[...]) * r[:, None]
    ps = jnp.concatenate([p1, p2], axis=1)
    ps_ref[...] = ps.reshape(MB, NCHUNK, CW).transpose(1, 0, 2)


def _stage_b(t1, r, b1, wmu, wls):
    return pl.pallas_call(
        _stage_b_body,
        grid=(GRID,),
        in_specs=[
            pl.BlockSpec((NCHUNK, MB, CW), lambda i: (0, i, 0)),
            pl.BlockSpec((1, 1, MB), lambda i: (i, 0, 0)),
            pl.BlockSpec((1, D_HID), lambda i: (0, 0)),
            pl.BlockSpec((D_HID, D_OUT), lambda i: (0, 0)),
            pl.BlockSpec((D_HID, D_OUT), lambda i: (0, 0)),
        ],
        out_specs=pl.BlockSpec((NCHUNK, MB, CW), lambda i: (0, i, 0)),
        out_shape=jax.ShapeDtypeStruct((NCHUNK, N, CW), jnp.float32),
    )(t1, r, b1, wmu, wls)


def _stage_c_body(t2_ref, r_ref, bmu_ref, bls_ref, mu_ref, ls_ref):
    t = t2_ref[...].transpose(1, 0, 2).reshape(MB, 2 * D_OUT)
    r = r_ref[0, 0, :]
    mu_ref[...] = t[:, :D_OUT] * r[:, None] + bmu_ref[0, :][None, :]
    ls_ref[...] = t[:, D_OUT:] * r[:, None] + bls_ref[0, :][None, :]


def _stage_c(t2, r, bmu, bls):
    return pl.pallas_call(
        _stage_c_body,
        grid=(GRID,),
        in_specs=[
            pl.BlockSpec((NCHUNK, MB, CW), lambda i: (0, i, 0)),
            pl.BlockSpec((1, 1, MB), lambda i: (i, 0, 0)),
            pl.BlockSpec((1, D_OUT), lambda i: (0, 0)),
            pl.BlockSpec((1, D_OUT), lambda i: (0, 0)),
        ],
        out_specs=[
            pl.BlockSpec((MB, D_OUT), lambda i: (i, 0)),
            pl.BlockSpec((MB, D_OUT), lambda i: (i, 0)),
        ],
        out_shape=[
            jax.ShapeDtypeStruct((N, D_OUT), jnp.float32),
            jax.ShapeDtypeStruct((N, D_OUT), jnp.float32),
        ],
    )(t2, r, bmu, bls)


def _deg_sc(dst32, ones, zeros):
    @functools.partial(
        pl.kernel,
        out_type=jax.ShapeDtypeStruct((2, N, 128), jnp.float32),
        mesh=plsc.VectorSubcoreMesh(**_SC_MESH),
        scratch_types=[
            pltpu.VMEM((EPAD // 32 // 128, 128), jnp.int32),
            pltpu.VMEM((128, 128), jnp.float32),
            pltpu.VMEM_SHARED((10008, 128), jnp.float32),
            pltpu.SemaphoreType.DMA,
        ],
    )
    def run(dst_hbm, ones_hbm, zeros_hbm, degp_hbm, dst_v, ones_v, acc, sem):
        c = lax.axis_index("c")
        s = lax.axis_index("s")
        w = c * 16 + s
        pltpu.sync_copy(dst_hbm.at[w], dst_v)
        pltpu.sync_copy(ones_hbm, ones_v)
        _striped_copy(s, lambda o, l: zeros_hbm.at[pl.ds(o, l)],
                      lambda o, l: acc.at[pl.ds(o, l)], total=10008)
        plsc.subcore_barrier()

        def body(b, carry):
            pltpu.async_copy(ones_v, acc.at[dst_v.at[b]], sem, add=True).wait()
            return carry
        lax.fori_loop(0, EPAD // 32 // 128, body, 0)

        plsc.subcore_barrier()
        _striped_copy(s, lambda o, l: acc.at[pl.ds(o, l)],
                      lambda o, l: degp_hbm.at[c, pl.ds(o, l)])

    return run(dst32, ones, zeros)


def _spmm_sc(table2d, src_flat, dst16):
    @functools.partial(
        pl.kernel,
        out_type=jax.ShapeDtypeStruct((NCHUNK * N, CW), jnp.float32),
        mesh=plsc.VectorSubcoreMesh(**_SC_MESH),
        scratch_types=[
            pltpu.VMEM((EPT,), jnp.int32),
            pltpu.VMEM((2, SEG, BT), jnp.int32),
            pltpu.VMEM((4, BT, CW), jnp.float32),
            pltpu.VMEM_SHARED((ACC_ROWS, CW), jnp.float32),
            pltpu.SemaphoreType.DMA((4,)),
            pltpu.SemaphoreType.DMA((4,)),
            pltpu.SemaphoreType.DMA((2,)),
        ],
    )
    def run(tab_hbm, src_hbm, dst_hbm, out_hbm, src_v, dst_v, rows, acc,
            gsem, ssem, dsem):
        c = lax.axis_index("c")
        s = lax.axis_index("s")
        pltpu.sync_copy(src_hbm.at[pl.ds(s * EPT, EPT)], src_v)

        def start_dseg(g):
            pltpu.async_copy(dst_hbm.at[s, pl.ds(g * SEG, SEG)],
                             dst_v.at[g % 2], dsem.at[g % 2])

        for p in range(2):
            delta = c * (2 * N) if p == 0 else N

            def adj(i, carry):
                ib = pl.multiple_of(i * 16, 16)
                src_v[pl.ds(ib, 16)] = src_v[pl.ds(ib, 16)] + delta
                return carry
            lax.fori_loop(0, EPT // 16, adj, 0)

            chunk = c * 2 + p
            base = chunk * N
            _striped_copy(
                s,
                lambda o, l: tab_hbm.at[pl.ds(pl.multiple_of(base + o, 8), l)],
                lambda o, l: acc.at[pl.ds(o, l)])
            plsc.subcore_barrier()

            def start_gather(b, r):
                bb = pl.multiple_of(b * BT, BT)
                pltpu.async_copy(tab_hbm.at[src_v.at[pl.ds(bb, BT)]],
                                 rows.at[r], gsem.at[r])

            start_dseg(0)
            start_dseg(1)
            for r in range(4):
                start_gather(r, r)
            for g in range(NB // SEG):
                q = g % 2
                pltpu.make_async_copy(dst_hbm.at[s, pl.ds(0, SEG)],
                                      dst_v.at[q], dsem.at[q]).wait()

                def inner(j4, carry):
                    for r in range(4):
                        jloc = j4 * 4 + r
                        b = g * SEG + jloc
                        pltpu.make_async_copy(
                            tab_hbm.at[pl.ds(0, BT)], rows.at[r],
                            gsem.at[r]).wait()
                        pltpu.async_copy(rows.at[r],
                                         acc.at[dst_v.at[q, jloc]],
                                         ssem.at[r], add=True).wait()

                        @pl.when(b + 4 < NB)
                        def _():
                            start_gather(b + 4, r)
                    return carry
                lax.fori_loop(0, SEG // 4, inner, 0)
                if g + 2 < NB // SEG:
                    start_dseg(g + 2)

            plsc.subcore_barrier()
            _striped_copy(
                s,
                lambda o, l: acc.at[pl.ds(o, l)],
                lambda o, l: out_hbm.at[pl.ds(pl.multiple_of(base + o, 8), l)])

    return run(table2d, src_flat, dst16)


def kernel(x, edge_index, W1, b1, W_mu, b_mu, W_ls, b_ls):
    src = edge_index[0].astype(jnp.int32)
    dst = edge_index[1].astype(jnp.int32)
    src_pad = jnp.concatenate([src, jnp.zeros((EPAD - E,), jnp.int32)])
    dst_pad = jnp.concatenate([dst, jnp.full((EPAD - E,), N, jnp.int32)])
    dst32 = dst_pad.reshape(32, EPAD // 32 // 128, 128)
    dst16 = dst_pad.reshape(16, NB, BT)
    ones = jnp.ones((128, 128), jnp.float32)
    zeros = jnp.zeros((10008, 128), jnp.float32)

    degp = _deg_sc(dst32, ones, zeros)
    gs, r = _stage_a(x, W1, degp)
    t1 = _spmm_sc(gs.reshape(NCHUNK * N, CW), src_pad, dst16)
    ps = _stage_b(t1.reshape(NCHUNK, N, CW), r, b1.reshape(1, -1),
                  W_mu, W_ls)
    t2 = _spmm_sc(ps.reshape(NCHUNK * N, CW), src_pad, dst16)
    mu, ls = _stage_c(t2.reshape(NCHUNK, N, CW), r, b_mu.reshape(1, -1),
                      b_ls.reshape(1, -1))
    return (mu, ls)


# speedup vs baseline: 7.4278x; 1.0239x over previous
import functools
import jax
import jax.numpy as jnp
from jax import lax
from jax.experimental import pallas as pl
from jax.experimental.pallas import tpu as pltpu
from jax.experimental.pallas import tpu_sc as plsc

N = 10000
D_IN = 256
D_HID = 512
D_OUT = 256
MB = 1000
GRID = N // MB
NCHUNK = 4
CW = 128

E = 160000
EPAD = 163840
EPT = EPAD // 16
BT = 64
NB = EPT // BT
SEG = 16
ACC_ROWS = 10008
ST_A = 632
ST_LAST = N - 15 * ST_A


def _striped_copy(s, mk_src, mk_dst, total=N):
    last = total - 15 * ST_A

    @pl.when(s < 15)
    def _():
        off = pl.multiple_of(s * ST_A, 8)
        pltpu.sync_copy(mk_src(off, ST_A), mk_dst(off, ST_A))

    @pl.when(s == 15)
    def _():
        pltpu.sync_copy(mk_src(15 * ST_A, last), mk_dst(15 * ST_A, last))

_SC_MESH = dict(core_axis_name="c", subcore_axis_name="s",
                num_cores=2, num_subcores=16)


def _dot(a, b):
    return lax.dot_general(a, b, (((1,), (0,)), ((), ())),
                           preferred_element_type=jnp.float32)


def _stage_a_body(x_ref, w1_ref, degp_ref, gs_ref, r_ref):
    deg = degp_ref[0, :, 0] + degp_ref[1, :, 0] + 1.0
    r = lax.rsqrt(deg)
    g = _dot(x_ref[...], w1_ref[...])
    gs = g * r[:, None]
    gs_ref[...] = gs.reshape(MB, NCHUNK, CW).transpose(1, 0, 2)
    r_ref[...] = r[None, None, :]


def _stage_a(x, w1, degp):
    return pl.pallas_call(
        _stage_a_body,
        grid=(GRID,),
        in_specs=[
            pl.BlockSpec((MB, D_IN), lambda i: (i, 0)),
            pl.BlockSpec((D_IN, D_HID), lambda i: (0, 0)),
            pl.BlockSpec((2, MB, 128), lambda i: (0, i, 0)),
        ],
        out_specs=[
            pl.BlockSpec((NCHUNK, MB, CW), lambda i: (0, i, 0)),
            pl.BlockSpec((1, 1, MB), lambda i: (i, 0, 0)),
        ],
        out_shape=[
            jax.ShapeDtypeStruct((NCHUNK, N, CW), jnp.float32),
            jax.ShapeDtypeStruct((GRID, 1, MB), jnp.float32),
        ],
    )(x, w1, degp)


def _stage_b_body(t1_ref, r_ref, b1_ref, wmu_ref, wls_ref, ps_ref):
    t = t1_ref[...].transpose(1, 0, 2).reshape(MB, D_HID)
    r = r_ref[0, 0, :]
    h = jnp.maximum(t * r[:, None] + b1_ref[0, :][None, :], 0.0)
    p1 = _dot(h, wmu_ref[...]) * r[:, None]
    p2 = _dot(h, wls_ref[...]) * r[:, None]
    ps = jnp.concatenate([p1, p2], axis=1)
    ps_ref[...] = ps.reshape(MB, NCHUNK, CW).transpose(1, 0, 2)


def _stage_b(t1, r, b1, wmu, wls):
    return pl.pallas_call(
        _stage_b_body,
        grid=(GRID,),
        in_specs=[
            pl.BlockSpec((NCHUNK, MB, CW), lambda i: (0, i, 0)),
            pl.BlockSpec((1, 1, MB), lambda i: (i, 0, 0)),
            pl.BlockSpec((1, D_HID), lambda i: (0, 0)),
            pl.BlockSpec((D_HID, D_OUT), lambda i: (0, 0)),
            pl.BlockSpec((D_HID, D_OUT), lambda i: (0, 0)),
        ],
        out_specs=pl.BlockSpec((NCHUNK, MB, CW), lambda i: (0, i, 0)),
        out_shape=jax.ShapeDtypeStruct((NCHUNK, N, CW), jnp.float32),
    )(t1, r, b1, wmu, wls)


def _stage_c_body(t2_ref, r_ref, bmu_ref, bls_ref, mu_ref, ls_ref):
    t = t2_ref[...].transpose(1, 0, 2).reshape(MB, 2 * D_OUT)
    r = r_ref[0, 0, :]
    mu_ref[...] = t[:, :D_OUT] * r[:, None] + bmu_ref[0, :][None, :]
    ls_ref[...] = t[:, D_OUT:] * r[:, None] + bls_ref[0, :][None, :]


def _stage_c(t2, r, bmu, bls):
    return pl.pallas_call(
        _stage_c_body,
        grid=(GRID,),
        in_specs=[
            pl.BlockSpec((NCHUNK, MB, CW), lambda i: (0, i, 0)),
            pl.BlockSpec((1, 1, MB), lambda i: (i, 0, 0)),
            pl.BlockSpec((1, D_OUT), lambda i: (0, 0)),
            pl.BlockSpec((1, D_OUT), lambda i: (0, 0)),
        ],
        out_specs=[
            pl.BlockSpec((MB, D_OUT), lambda i: (i, 0)),
            pl.BlockSpec((MB, D_OUT), lambda i: (i, 0)),
        ],
        out_shape=[
            jax.ShapeDtypeStruct((N, D_OUT), jnp.float32),
            jax.ShapeDtypeStruct((N, D_OUT), jnp.float32),
        ],
    )(t2, r, bmu, bls)


def _deg_sc(dst32, ones, zeros):
    @functools.partial(
        pl.kernel,
        out_type=jax.ShapeDtypeStruct((2, N, 128), jnp.float32),
        mesh=plsc.VectorSubcoreMesh(**_SC_MESH),
        scratch_types=[
            pltpu.VMEM((EPAD // 32 // 128, 128), jnp.int32),
            pltpu.VMEM((128, 128), jnp.float32),
            pltpu.VMEM_SHARED((10008, 128), jnp.float32),
            pltpu.SemaphoreType.DMA,
        ],
    )
    def run(dst_hbm, ones_hbm, zeros_hbm, degp_hbm, dst_v, ones_v, acc, sem):
        c = lax.axis_index("c")
        s = lax.axis_index("s")
        w = c * 16 + s
        pltpu.sync_copy(dst_hbm.at[w], dst_v)
        pltpu.sync_copy(ones_hbm, ones_v)
        _striped_copy(s, lambda o, l: zeros_hbm.at[pl.ds(o, l)],
                      lambda o, l: acc.at[pl.ds(o, l)], total=10008)
        plsc.subcore_barrier()

        def body(b, carry):
            pltpu.async_copy(ones_v, acc.at[dst_v.at[b]], sem, add=True).wait()
            return carry
        lax.fori_loop(0, EPAD // 32 // 128, body, 0)

        plsc.subcore_barrier()
        _striped_copy(s, lambda o, l: acc.at[pl.ds(o, l)],
                      lambda o, l: degp_hbm.at[c, pl.ds(o, l)])

    return run(dst32, ones, zeros)


def _spmm_sc(table2d, src_flat, dst16):
    @functools.partial(
        pl.kernel,
        out_type=jax.ShapeDtypeStruct((NCHUNK * N, CW), jnp.float32),
        mesh=plsc.VectorSubcoreMesh(**_SC_MESH),
        scratch_types=[
            pltpu.VMEM((EPT,), jnp.int32),
            pltpu.VMEM((2, SEG, BT), jnp.int32),
            pltpu.VMEM((4, BT, CW), jnp.float32),
            pltpu.VMEM_SHARED((ACC_ROWS, CW), jnp.float32),
            pltpu.SemaphoreType.DMA((4,)),
            pltpu.SemaphoreType.DMA((4,)),
            pltpu.SemaphoreType.DMA((2,)),
        ],
    )
    def run(tab_hbm, src_hbm, dst_hbm, out_hbm, src_v, dst_v, rows, acc,
            gsem, ssem, dsem):
        c = lax.axis_index("c")
        s = lax.axis_index("s")
        pltpu.sync_copy(src_hbm.at[pl.ds(s * EPT, EPT)], src_v)

        def start_dseg(g):
            pltpu.async_copy(dst_hbm.at[s, pl.ds(g * SEG, SEG)],
                             dst_v.at[g % 2], dsem.at[g % 2])

        for p in range(2):
            delta = c * (2 * N) if p == 0 else N

            def adj(i, carry):
                ib = pl.multiple_of(i * 16, 16)
                src_v[pl.ds(ib, 16)] = src_v[pl.ds(ib, 16)] + delta
                return carry
            lax.fori_loop(0, EPT // 16, adj, 0)

            chunk = c * 2 + p
            base = chunk * N
            _striped_copy(
                s,
                lambda o, l: tab_hbm.at[pl.ds(pl.multiple_of(base + o, 8), l)],
                lambda o, l: acc.at[pl.ds(o, l)])
            plsc.subcore_barrier()

            def start_gather(b, r):
                bb = pl.multiple_of(b * BT, BT)
                pltpu.async_copy(tab_hbm.at[src_v.at[pl.ds(bb, BT)]],
                                 rows.at[r], gsem.at[r])

            start_dseg(0)
            start_dseg(1)
            for r in range(4):
                start_gather(r, r)
            for g in range(NB // SEG):
                q = g % 2
                pltpu.make_async_copy(dst_hbm.at[s, pl.ds(0, SEG)],
                                      dst_v.at[q], dsem.at[q]).wait()

                def inner(j4, carry):
                    for r in range(4):
                        jloc = j4 * 4 + r
                        b = g * SEG + jloc
                        pltpu.make_async_copy(
                            tab_hbm.at[pl.ds(0, BT)], rows.at[r],
                            gsem.at[r]).wait()
                        pltpu.async_copy(rows.at[r],
                                         acc.at[dst_v.at[q, jloc]],
                                         ssem.at[r], add=True).wait()

                        @pl.when(b + 4 < NB)
                        def _():
                            start_gather(b + 4, r)
                    return carry
                lax.fori_loop(0, SEG // 4, inner, 0)
                if g + 2 < NB // SEG:
                    start_dseg(g + 2)

            plsc.subcore_barrier()
            _striped_copy(
                s,
                lambda o, l: acc.at[pl.ds(o, l)],
                lambda o, l: out_hbm.at[pl.ds(pl.multiple_of(base + o, 8), l)])

    return run(table2d, src_flat, dst16)


def kernel(x, edge_index, W1, b1, W_mu, b_mu, W_ls, b_ls):
    src = edge_index[0].astype(jnp.int32)
    dst = edge_index[1].astype(jnp.int32)
    src_pad = jnp.concatenate([src, jnp.zeros((EPAD - E,), jnp.int32)])
    dst_pad = jnp.concatenate([dst, jnp.full((EPAD - E,), N, jnp.int32)])
    dst32 = dst_pad.reshape(32, EPAD // 32 // 128, 128)
    dst16 = dst_pad.reshape(16, NB, BT)
    ones = jnp.ones((128, 128), jnp.float32)
    zeros = jnp.zeros((10008, 128), jnp.float32)

    degp = _deg_sc(dst32, ones, zeros)
    gs, r = _stage_a(x, W1, degp)
    t1 = _spmm_sc(gs.reshape(NCHUNK * N, CW), src_pad, dst16)
    ps = _stage_b(t1.reshape(NCHUNK, N, CW), r, b1.reshape(1, -1),
                  W_mu, W_ls)
    t2 = _spmm_sc(ps.reshape(NCHUNK * N, CW), src_pad, dst16)
    mu, ls = _stage_c(t2.reshape(NCHUNK, N, CW), r, b_mu.reshape(1, -1),
                      b_ls.reshape(1, -1))
    return (mu, ls)
